# baseline (speedup 1.0000x reference)
import functools

import jax
import jax.numpy as jnp
from jax.experimental import pallas as pl
from jax.experimental.pallas import tpu as pltpu

N, H, W = 16, 224, 224
C = 96
HBT = 224


def _tc_body(x_ref, o_ref):
    x = x_ref[0]
    cio = jax.lax.broadcasted_iota(jnp.int32, (C, HBT, W), 0)
    o_ref[0] = jnp.where(cio == x[None], 1.0, 0.0).astype(jnp.float32)


def tc_onehot(x):
    n, h, w = x.shape
    grid = (n, h // HBT)
    return pl.pallas_call(
        _tc_body,
        grid=grid,
        in_specs=[
            pl.BlockSpec((1, HBT, w), lambda i, j: (i, j, 0)),
        ],
        out_specs=pl.BlockSpec((1, C, HBT, w), lambda i, j: (i, 0, j, 0)),
        out_shape=jax.ShapeDtypeStruct((n, C, h, w), jnp.float32),
    )(x)


def kernel(x):
    return tc_onehot(x)
